# Initial kernel scaffold; baseline (speedup 1.0000x reference)
import jax
import jax.numpy as jnp
from jax.experimental import pallas as pl


def kernel(input_x, mask, temp_queries, prev_exists, W_proj, b_proj, W_agg, b_agg):
    raise NotImplementedError("write your pallas kernel here")


# speedup vs baseline: 1.5526x; 1.5526x over previous
import functools

import jax
import jax.numpy as jnp
from jax.experimental import pallas as pl

B, H, W, C = 16, 32, 32, 768
Q, DQ = 256, 256
N = H * W
KEEP = N // 2


def _selector_kernel(x_ref, m_ref, q_ref, wp_ref, bp_ref, wa_ref, ba_ref,
                     ks_ref, ds_ref, ki_ref, di_ref, nm_ref, sc_ref):
    x = x_ref[0] * m_ref[0]
    dn = (((1,), (1,)), ((), ()))
    wp = wp_ref[...]
    xp = (jax.lax.dot_general(x[:, :256], wp[:, :256], dn,
                              preferred_element_type=jnp.float32)
          + jax.lax.dot_general(x[:, 256:512], wp[:, 256:512], dn,
                                preferred_element_type=jnp.float32)
          + jax.lax.dot_general(x[:, 512:], wp[:, 512:], dn,
                                preferred_element_type=jnp.float32)) + bp_ref[...]
    att = jax.lax.dot_general(xp, q_ref[0], dn,
                              preferred_element_type=jnp.float32) * (DQ ** -0.5)
    logits = jax.lax.dot_general(att, wa_ref[...], dn,
                                 preferred_element_type=jnp.float32) + ba_ref[...]
    mx = jnp.max(logits, axis=-1, keepdims=True)
    sh = logits - mx
    lsm = sh - jnp.log(jnp.sum(jnp.exp(sh), axis=-1, keepdims=True))
    s_col = lsm[:, 0:1]

    ii = jax.lax.broadcasted_iota(jnp.int32, (N, N), 0)
    jj = jax.lax.broadcasted_iota(jnp.int32, (N, N), 1)
    s_row = jnp.sum(jnp.where(ii == jj, s_col, 0.0), axis=0, keepdims=True)

    si = s_col
    sj = s_row
    g = jnp.where((si > sj) | ((si == sj) & (ii < jj)), 1, 0)
    rank_row = jnp.sum(g, axis=0, keepdims=True)
    rank_col = (N - 1) - jnp.sum(g, axis=1, keepdims=True)

    sel = rank_col == jj
    svals = jnp.sum(jnp.where(sel, s_col, 0.0), axis=0, keepdims=True)
    sidx = jnp.sum(jnp.where(sel, ii, 0), axis=0, keepdims=True)

    ks_ref[...] = svals[:, :KEEP].reshape(1, 1, KEEP)
    ds_ref[...] = svals[:, KEEP:].reshape(1, 1, KEEP)
    ki_ref[...] = sidx[:, :KEEP].reshape(1, 1, KEEP)
    di_ref[...] = sidx[:, KEEP:].reshape(1, 1, KEEP)
    nm_ref[...] = jnp.where(rank_row < KEEP, 1.0, 0.0).reshape(1, 1, N)
    sc_ref[...] = s_row.reshape(1, 1, N)


@functools.partial(jax.jit, static_argnames=("interpret",))
def kernel(input_x, mask, temp_queries, prev_exists, W_proj, b_proj, W_agg,
           b_agg, interpret=False):
    b = input_x.shape[0]
    x = input_x.reshape(b, N, C)
    m = mask.reshape(b, N, 1)
    bp = b_proj.reshape(1, DQ)
    ba = b_agg.reshape(1, 2)

    row_spec = lambda n: pl.BlockSpec((1, 1, n), lambda i: (i, 0, 0))
    full = lambda a: pl.BlockSpec(a.shape, lambda i: (0,) * a.ndim)

    out_shapes = [
        jax.ShapeDtypeStruct((b, 1, KEEP), jnp.float32),
        jax.ShapeDtypeStruct((b, 1, KEEP), jnp.float32),
        jax.ShapeDtypeStruct((b, 1, KEEP), jnp.int32),
        jax.ShapeDtypeStruct((b, 1, KEEP), jnp.int32),
        jax.ShapeDtypeStruct((b, 1, N), jnp.float32),
        jax.ShapeDtypeStruct((b, 1, N), jnp.float32),
    ]
    outs = pl.pallas_call(
        _selector_kernel,
        grid=(b,),
        in_specs=[
            pl.BlockSpec((1, N, C), lambda i: (i, 0, 0)),
            pl.BlockSpec((1, N, 1), lambda i: (i, 0, 0)),
            pl.BlockSpec((1, Q, DQ), lambda i: (i, 0, 0)),
            full(W_proj), full(bp), full(W_agg), full(ba),
        ],
        out_specs=[row_spec(KEEP), row_spec(KEEP), row_spec(KEEP),
                   row_spec(KEEP), row_spec(N), row_spec(N)],
        out_shape=out_shapes,
        interpret=interpret,
    )(x, m, temp_queries, W_proj, bp, W_agg, ba)
    ks, ds, ki, di, nm, sc = outs
    return (ks.reshape(b, KEEP), ds.reshape(b, KEEP), ki.reshape(b, KEEP),
            di.reshape(b, KEEP), nm.reshape(b, H, W, 1), sc.reshape(b, H, W))
